# baseline (device time: 16144 ns/iter reference)
import jax
import jax.numpy as jnp
from jax import lax
from jax.experimental import pallas as pl
from jax.experimental.pallas import tpu as pltpu

M = 1024
N = 1024
HALF = N // 2
Q = N // 4
K = 8
CM = M // K


def kernel(x):
    def body(x_hbm, out_hbm, xp_f32, me_f32, send_x, recv_x, fbuf,
             xp_sems, me_sems, st_sems,
             x_send_sems, x_recv_sems, y_send_sems, y_recv_sems):
        my_x = lax.axis_index("x")
        my_y = lax.axis_index("y")
        x_peer = (1 - my_x, my_y)
        y_peer = (my_x, 1 - my_y)
        c_me = my_x * HALF + my_y * Q
        c_xp = (1 - my_x) * HALF + my_y * Q
        off = my_y * Q

        def fetch(k, col, dst, sems):
            rows = pl.ds(k * CM, CM)
            return pltpu.make_async_copy(
                x_hbm.at[0, rows, pl.ds(col, Q)], dst.at[rows], sems.at[k]
            )

        def store(k):
            rows = pl.ds(k * CM, CM)
            return pltpu.make_async_copy(
                fbuf.at[rows], out_hbm.at[rows, pl.ds(off, Q)], st_sems.at[k]
            )

        def x_rdma(k):
            rows = pl.ds(k * CM, CM)
            return pltpu.make_async_remote_copy(
                src_ref=send_x.at[rows],
                dst_ref=recv_x.at[rows],
                send_sem=x_send_sems.at[k],
                recv_sem=x_recv_sems.at[k],
                device_id=x_peer,
                device_id_type=pl.DeviceIdType.MESH,
            )

        def y_rdma(k):
            rows = pl.ds(k * CM, CM)
            return pltpu.make_async_remote_copy(
                src_ref=fbuf.at[rows],
                dst_ref=out_hbm.at[rows, pl.ds(off, Q)],
                send_sem=y_send_sems.at[k],
                recv_sem=y_recv_sems.at[k],
                device_id=y_peer,
                device_id_type=pl.DeviceIdType.MESH,
            )

        barrier = pltpu.get_barrier_semaphore()
        for peer in (x_peer, y_peer):
            pl.semaphore_signal(
                barrier, inc=1, device_id=peer,
                device_id_type=pl.DeviceIdType.MESH,
            )
        for k in range(K):
            fetch(k, c_xp, xp_f32, xp_sems).start()
        for k in range(K):
            fetch(k, c_me, me_f32, me_sems).start()
        pl.semaphore_wait(barrier, 2)

        for k in range(K):
            rows = pl.ds(k * CM, CM)
            fetch(k, c_xp, xp_f32, xp_sems).wait()
            send_x[rows, :] = xp_f32[rows, :].astype(jnp.bfloat16)
            x_rdma(k).start()

        for k in range(K):
            rows = pl.ds(k * CM, CM)
            x_rdma(k).wait_recv()
            fetch(k, c_me, me_f32, me_sems).wait()
            fbuf[rows, :] = me_f32[rows, :].astype(jnp.bfloat16) + recv_x[rows, :]
            store(k).start()
            y_rdma(k).start()

        for k in range(K):
            y_rdma(k).wait_recv()

        for k in range(K):
            store(k).wait()
            x_rdma(k).wait_send()
            y_rdma(k).wait_send()

    return pl.pallas_call(
        body,
        out_shape=jax.ShapeDtypeStruct((M, HALF), jnp.bfloat16),
        in_specs=[pl.BlockSpec(memory_space=pltpu.MemorySpace.HBM)],
        out_specs=pl.BlockSpec(memory_space=pltpu.MemorySpace.HBM),
        scratch_shapes=[
            pltpu.VMEM((M, Q), jnp.float32),
            pltpu.VMEM((M, Q), jnp.float32),
            pltpu.VMEM((M, Q), jnp.bfloat16),
            pltpu.VMEM((M, Q), jnp.bfloat16),
            pltpu.VMEM((M, Q), jnp.bfloat16),
            pltpu.SemaphoreType.DMA((K,)),
            pltpu.SemaphoreType.DMA((K,)),
            pltpu.SemaphoreType.DMA((K,)),
            pltpu.SemaphoreType.DMA((K,)),
            pltpu.SemaphoreType.DMA((K,)),
            pltpu.SemaphoreType.DMA((K,)),
            pltpu.SemaphoreType.DMA((K,)),
        ],
        compiler_params=pltpu.CompilerParams(collective_id=0),
    )(x)
